# Initial kernel scaffold; baseline (speedup 1.0000x reference)
import jax
import jax.numpy as jnp
from jax.experimental import pallas as pl


def kernel(input, params):
    raise NotImplementedError("write your pallas kernel here")


# speedup vs baseline: 13.6951x; 13.6951x over previous
import functools

import jax
import jax.numpy as jnp
from jax.experimental import pallas as pl

_R2 = 0.2 * 0.2
_K = 32
_L = 4
_NP = 128
_N = 1024
_D = 512
_HEADS = 8
_DH = 64
_NEG = -1e30


def _dg(a, b):
    return jax.lax.dot_general(a, b, (((1,), (0,)), ((), ())),
                               preferred_element_type=jnp.float32)


def _dgt(a, b):
    return jax.lax.dot_general(a, b, (((1,), (1,)), ((), ())),
                               preferred_element_type=jnp.float32)


def _fps_body(x_ref, y_ref, z_ref, ax_ref, ay_ref, az_ref):
    x = x_ref[...]
    y = y_ref[...]
    z = z_ref[...]
    P = x.shape[0]
    lin = (jax.lax.broadcasted_iota(jnp.int32, (1, 8, 128), 1) * 128 +
           jax.lax.broadcasted_iota(jnp.int32, (1, 8, 128), 2))
    col = jax.lax.broadcasted_iota(jnp.int32, (1, _NP), 1)

    def step(s, carry):
        dists, far, ax, ay, az = carry
        m = lin == far
        cx = jnp.sum(jnp.where(m, x, 0.), axis=(1, 2), keepdims=True)
        cy = jnp.sum(jnp.where(m, y, 0.), axis=(1, 2), keepdims=True)
        cz = jnp.sum(jnp.where(m, z, 0.), axis=(1, 2), keepdims=True)
        ax = jnp.where(col == s, cx[:, :, 0], ax)
        ay = jnp.where(col == s, cy[:, :, 0], ay)
        az = jnp.where(col == s, cz[:, :, 0], az)
        d = (x - cx) ** 2 + (y - cy) ** 2 + (z - cz) ** 2
        dists = jnp.minimum(dists, d)
        mx = jnp.max(dists, axis=(1, 2), keepdims=True)
        cand = jnp.where(dists == mx, lin, jnp.int32(1 << 30))
        far = jnp.min(cand, axis=(1, 2), keepdims=True)
        return dists, far, ax, ay, az

    init = (jnp.full((P, 8, 128), 1e10, jnp.float32),
            jnp.zeros((P, 1, 1), jnp.int32),
            jnp.zeros((P, _NP), jnp.float32),
            jnp.zeros((P, _NP), jnp.float32),
            jnp.zeros((P, _NP), jnp.float32))
    _, _, ax, ay, az = jax.lax.fori_loop(0, _NP, step, init)
    ax_ref[...] = ax
    ay_ref[...] = ay
    az_ref[...] = az


def _group_body(a_ref, x0_ref, x1_ref, x2_ref, t0_ref, t1_ref, t2_ref,
                w3_ref, w4_ref, o_ref):
    a = a_ref[0]
    w3 = w3_ref[...]
    w4 = w4_ref[...]
    ax = a[:, 0:1]
    ay = a[:, 1:2]
    az = a[:, 2:3]
    r = jax.lax.broadcasted_iota(jnp.int32, (_N, _N), 0)
    c = jax.lax.broadcasted_iota(jnp.int32, (_N, _N), 1)
    tri = (r <= c).astype(jnp.float32)
    kio3 = jax.lax.broadcasted_iota(jnp.int32, (1, _K, 1), 1).astype(jnp.float32)
    lastcol = jax.lax.broadcasted_iota(jnp.int32, (_NP, _N), 1) == _N - 1
    arep = jnp.broadcast_to(a[:, None, :], (_NP, _K, 3)).reshape(_NP * _K, 3)

    acc = jnp.full((_NP, _D), _NEG, jnp.float32)
    for di, (xr, tr) in zip((-1, 0, 1),
                            ((x0_ref, t0_ref), (x1_ref, t1_ref),
                             (x2_ref, t2_ref))):
        p = xr[0]
        pT = tr[0]
        px = pT[0:1, :]
        py = pT[1:2, :]
        pz = pT[2:3, :]
        d2 = (ax - px) ** 2 + (ay - py) ** 2 + (az - pz) ** 2
        mask = d2 < _R2
        mf = mask.astype(jnp.float32)
        rank_incl = _dg(mf, tri)
        rank_ex = rank_incl - mf
        count = rank_incl[:, _N - 1:_N]
        sel = mask & (rank_ex < float(_K))
        sel = sel | (lastcol & (count == 0.))
        o2 = (sel[:, None, :] & (rank_ex[:, None, :] == kio3))
        o2 = o2.astype(jnp.float32).reshape(_NP * _K, _N)
        g3 = _dg(o2, p)
        disp = g3 - arep
        fk = _dg(disp, w3).reshape(_NP, _K, _D)
        validk = kio3 < jnp.maximum(count[:, :, None], 1.)
        fi = jnp.max(jnp.where(validk, fk, _NEG), axis=1)
        acc = jnp.maximum(acc, fi + float(di) * w4)
    o_ref[0] = acc


def _ln(x, g, b):
    mu = jnp.mean(x, axis=-1, keepdims=True)
    var = jnp.mean((x - mu) ** 2, axis=-1, keepdims=True)
    return (x - mu) / jnp.sqrt(var + 1e-5) * g + b


def _tx_body(x_ref, pos_ref, lg_ref, lb_ref, wqkv_ref, wsp_ref, wout_ref,
             bout_ref, fg_ref, fb_ref, w1_ref, b1_ref, w2_ref, b2_ref,
             hg_ref, hb_ref, hw1_ref, hb1_ref, hw2_ref, hb2_ref, o_ref):
    x = x_ref[0]
    pos = pos_ref[0]
    scale = _DH ** -0.5
    for li in range(2):
        h = _ln(x, lg_ref[li], lb_ref[li])
        qkv = _dg(h, wqkv_ref[li])
        wsp = wsp_ref[li]
        outs = []
        for hh in range(_HEADS):
            q = qkv[:, hh * _DH:(hh + 1) * _DH]
            k = qkv[:, _D + hh * _DH:_D + (hh + 1) * _DH]
            v = qkv[:, 2 * _D + hh * _DH:2 * _D + (hh + 1) * _DH]
            qs = _dgt(q, wsp)
            dots = (_dgt(q, k) - _dgt(qs, pos)) * scale
            mx = jnp.max(dots, axis=-1, keepdims=True)
            e = jnp.exp(dots - mx)
            attn = e / jnp.sum(e, axis=-1, keepdims=True)
            outs.append(_dg(attn, v))
        o = jnp.concatenate(outs, axis=1)
        x = _dg(o, wout_ref[li]) + bout_ref[li] + x
        h2 = _ln(x, fg_ref[li], fb_ref[li])
        f = jax.nn.gelu(_dg(h2, w1_ref[li]) + b1_ref[li])
        x = _dg(f, w2_ref[li]) + b2_ref[li] + x
    pooled = jnp.max(x, axis=0, keepdims=True)
    hh_ = _ln(pooled, hg_ref[...], hb_ref[...])
    g1 = jax.nn.gelu(_dg(hh_, hw1_ref[...]) + hb1_ref[...])
    o_ref[0] = _dg(g1, hw2_ref[...]) + hb2_ref[...]


def kernel(input, params):
    xyzs = input
    B, L, N, _ = xyzs.shape
    BL = B * L
    xr = xyzs.reshape(BL, N, 3)
    x8 = xr[:, :, 0].reshape(BL, 8, 128)
    y8 = xr[:, :, 1].reshape(BL, 8, 128)
    z8 = xr[:, :, 2].reshape(BL, 8, 128)

    s_anchor = jax.ShapeDtypeStruct((BL, _NP), jnp.float32)
    ax, ay, az = pl.pallas_call(
        _fps_body,
        out_shape=[s_anchor, s_anchor, s_anchor],
    )(x8, y8, z8)
    anchors = jnp.stack([ax, ay, az], axis=-1)

    xT = jnp.swapaxes(xr, 1, 2)
    w = params['conv_d_w']
    w3 = jnp.swapaxes(w[:, :3], 0, 1)
    w4 = w[:, 3].reshape(1, _D)

    def fmap(di):
        def imap(g):
            b = g // L
            t = g % L
            return (b * L + jnp.clip(t + di, 0, L - 1), 0, 0)
        return imap

    feats = pl.pallas_call(
        _group_body,
        grid=(BL,),
        in_specs=[
            pl.BlockSpec((1, _NP, 3), lambda g: (g, 0, 0)),
            pl.BlockSpec((1, N, 3), fmap(-1)),
            pl.BlockSpec((1, N, 3), fmap(0)),
            pl.BlockSpec((1, N, 3), fmap(1)),
            pl.BlockSpec((1, 3, N), fmap(-1)),
            pl.BlockSpec((1, 3, N), fmap(0)),
            pl.BlockSpec((1, 3, N), fmap(1)),
            pl.BlockSpec((3, _D), lambda g: (0, 0)),
            pl.BlockSpec((1, _D), lambda g: (0, 0)),
        ],
        out_specs=pl.BlockSpec((1, _NP, _D), lambda g: (g, 0, 0)),
        out_shape=jax.ShapeDtypeStruct((BL, _NP, _D), jnp.float32),
    )(anchors, xr, xr, xr, xT, xT, xT, w3, w4)

    lp = params['layers']
    st = lambda nm: jnp.stack([l[nm] for l in lp])
    st1 = lambda nm: jnp.stack([l[nm].reshape(1, -1) for l in lp])
    lg, lb = st1('attn_norm_g'), st1('attn_norm_b')
    wqkv, wsp = st('w_qkv'), st('w_spatial')
    wout, bout = st('w_out'), st1('b_out')
    fg, fb = st1('ff_norm_g'), st1('ff_norm_b')
    w1, b1 = st('w1'), st1('b1')
    w2, b2 = st('w2'), st1('b2')
    hg = params['head_norm_g'].reshape(1, _D)
    hb = params['head_norm_b'].reshape(1, _D)
    hw1, hb1 = params['head_w1'], params['head_b1'].reshape(1, -1)
    hw2, hb2 = params['head_w2'], params['head_b2'].reshape(1, -1)
    ncls = hw2.shape[1]

    T = L * _NP
    full = lambda arr: pl.BlockSpec(arr.shape, lambda b: (0,) * arr.ndim)
    logits = pl.pallas_call(
        _tx_body,
        grid=(B,),
        in_specs=[
            pl.BlockSpec((1, T, _D), lambda b: (b, 0, 0)),
            pl.BlockSpec((1, T, 3), lambda b: (b, 0, 0)),
            full(lg), full(lb), full(wqkv), full(wsp), full(wout), full(bout),
            full(fg), full(fb), full(w1), full(b1), full(w2), full(b2),
            full(hg), full(hb), full(hw1), full(hb1), full(hw2), full(hb2),
        ],
        out_specs=pl.BlockSpec((1, 1, ncls), lambda b: (b, 0, 0)),
        out_shape=jax.ShapeDtypeStruct((B, 1, ncls), jnp.float32),
    )(feats.reshape(B, T, _D), anchors.reshape(B, T, 3),
      lg, lb, wqkv, wsp, wout, bout, fg, fb, w1, b1, w2, b2,
      hg, hb, hw1, hb1, hw2, hb2)
    return logits.reshape(B, ncls)


# speedup vs baseline: 14.9010x; 1.0881x over previous
import functools

import jax
import jax.numpy as jnp
from jax.experimental import pallas as pl

_R2 = 0.2 * 0.2
_K = 32
_L = 4
_NP = 128
_N = 1024
_D = 512
_HEADS = 8
_DH = 64
_NEG = -1e30


def _dg(a, b):
    return jax.lax.dot_general(a, b, (((1,), (0,)), ((), ())),
                               preferred_element_type=jnp.float32)


def _dgt(a, b):
    return jax.lax.dot_general(a, b, (((1,), (1,)), ((), ())),
                               preferred_element_type=jnp.float32)


def _fps_body(x_ref, y_ref, z_ref, ax_ref, ay_ref, az_ref):
    x = x_ref[...]
    y = y_ref[...]
    z = z_ref[...]
    P = x.shape[0]
    lin = (jax.lax.broadcasted_iota(jnp.int32, (1, 8, 128), 1) * 128 +
           jax.lax.broadcasted_iota(jnp.int32, (1, 8, 128), 2))
    col = jax.lax.broadcasted_iota(jnp.int32, (1, _NP), 1)

    def step(s, carry):
        dists, far, ax, ay, az = carry
        m = lin == far
        cx = jnp.sum(jnp.where(m, x, 0.), axis=(1, 2), keepdims=True)
        cy = jnp.sum(jnp.where(m, y, 0.), axis=(1, 2), keepdims=True)
        cz = jnp.sum(jnp.where(m, z, 0.), axis=(1, 2), keepdims=True)
        ax = jnp.where(col == s, cx[:, :, 0], ax)
        ay = jnp.where(col == s, cy[:, :, 0], ay)
        az = jnp.where(col == s, cz[:, :, 0], az)
        d = (x - cx) ** 2 + (y - cy) ** 2 + (z - cz) ** 2
        dists = jnp.minimum(dists, d)
        mx = jnp.max(dists, axis=(1, 2), keepdims=True)
        cand = jnp.where(dists == mx, lin, jnp.int32(1 << 30))
        far = jnp.min(cand, axis=(1, 2), keepdims=True)
        return dists, far, ax, ay, az

    init = (jnp.full((P, 8, 128), 1e10, jnp.float32),
            jnp.zeros((P, 1, 1), jnp.int32),
            jnp.zeros((P, _NP), jnp.float32),
            jnp.zeros((P, _NP), jnp.float32),
            jnp.zeros((P, _NP), jnp.float32))
    _, _, ax, ay, az = jax.lax.fori_loop(0, _NP, step, init)
    ax_ref[...] = ax
    ay_ref[...] = ay
    az_ref[...] = az


def _group_body(a_ref, x0_ref, x1_ref, x2_ref, t0_ref, t1_ref, t2_ref,
                w3_ref, w4_ref, o_ref):
    a = a_ref[0]
    w3 = w3_ref[...]
    w4 = w4_ref[...]
    ax = a[:, 0:1]
    ay = a[:, 1:2]
    az = a[:, 2:3]
    r = jax.lax.broadcasted_iota(jnp.int32, (_N, _N), 0)
    c = jax.lax.broadcasted_iota(jnp.int32, (_N, _N), 1)
    tri = (r <= c).astype(jnp.bfloat16)
    kio3 = jax.lax.broadcasted_iota(jnp.int32, (1, _K, 1), 1).astype(jnp.float32)
    lastcol = jax.lax.broadcasted_iota(jnp.int32, (_NP, _N), 1) == _N - 1
    arep = jnp.broadcast_to(a[:, None, :], (_NP, _K, 3)).reshape(_NP * _K, 3)

    acc = jnp.full((_NP, _D), _NEG, jnp.float32)
    for di, (xr, tr) in zip((-1, 0, 1),
                            ((x0_ref, t0_ref), (x1_ref, t1_ref),
                             (x2_ref, t2_ref))):
        p = xr[0]
        pT = tr[0]
        px = pT[0:1, :]
        py = pT[1:2, :]
        pz = pT[2:3, :]
        d2 = (ax - px) ** 2 + (ay - py) ** 2 + (az - pz) ** 2
        mask = d2 < _R2
        mf = mask.astype(jnp.float32)
        rank_incl = jax.lax.dot_general(
            mask.astype(jnp.bfloat16), tri, (((1,), (0,)), ((), ())),
            preferred_element_type=jnp.float32)
        rank_ex = rank_incl - mf
        count = rank_incl[:, _N - 1:_N]
        sel = mask & (rank_ex < float(_K))
        sel = sel | (lastcol & (count == 0.))
        o2 = (sel[:, None, :] & (rank_ex[:, None, :] == kio3))
        o2 = o2.astype(jnp.float32).reshape(_NP * _K, _N)
        g3 = _dg(o2, p)
        disp = g3 - arep
        fk = _dg(disp, w3).reshape(_NP, _K, _D)
        validk = kio3 < jnp.maximum(count[:, :, None], 1.)
        fi = jnp.max(jnp.where(validk, fk, _NEG), axis=1)
        acc = jnp.maximum(acc, fi + float(di) * w4)
    o_ref[0] = acc


def _ln(x, g, b):
    mu = jnp.mean(x, axis=-1, keepdims=True)
    var = jnp.mean((x - mu) ** 2, axis=-1, keepdims=True)
    return (x - mu) / jnp.sqrt(var + 1e-5) * g + b


def _tx_body(*refs):
    x_ref, pos_ref = refs[0], refs[1]
    hg_ref, hb_ref, hw1_ref, hb1_ref, hw2_ref, hb2_ref = refs[-7:-1]
    o_ref = refs[-1]
    x = x_ref[0]
    pos = pos_ref[0]
    scale = _DH ** -0.5
    for li in range(2):
        (lg_ref, lb_ref, wqkv_ref, wsp_ref, wout_ref, bout_ref, fg_ref,
         fb_ref, w1_ref, b1_ref, w2_ref, b2_ref) = refs[2 + 12 * li:
                                                        2 + 12 * (li + 1)]
        h = _ln(x, lg_ref[...], lb_ref[...])
        qkv = _dg(h, wqkv_ref[...])
        wsp = wsp_ref[...]
        outs = []
        for hh in range(_HEADS):
            q = qkv[:, hh * _DH:(hh + 1) * _DH]
            k = qkv[:, _D + hh * _DH:_D + (hh + 1) * _DH]
            v = qkv[:, 2 * _D + hh * _DH:2 * _D + (hh + 1) * _DH]
            qs = _dgt(q, wsp)
            dots = (_dgt(q, k) - _dgt(qs, pos)) * scale
            mx = jnp.max(dots, axis=-1, keepdims=True)
            e = jnp.exp(dots - mx)
            attn = e / jnp.sum(e, axis=-1, keepdims=True)
            outs.append(_dg(attn, v))
        o = jnp.concatenate(outs, axis=1)
        x = _dg(o, wout_ref[...]) + bout_ref[...] + x
        h2 = _ln(x, fg_ref[...], fb_ref[...])
        f = jax.nn.gelu(_dg(h2, w1_ref[...]) + b1_ref[...])
        x = _dg(f, w2_ref[...]) + b2_ref[...] + x
    pooled = jnp.max(x, axis=0, keepdims=True)
    hh_ = _ln(pooled, hg_ref[...], hb_ref[...])
    g1 = jax.nn.gelu(_dg(hh_, hw1_ref[...]) + hb1_ref[...])
    o_ref[0] = _dg(g1, hw2_ref[...]) + hb2_ref[...]


def kernel(input, params):
    xyzs = input
    B, L, N, _ = xyzs.shape
    BL = B * L
    xr = xyzs.reshape(BL, N, 3)
    x8 = xr[:, :, 0].reshape(BL, 8, 128)
    y8 = xr[:, :, 1].reshape(BL, 8, 128)
    z8 = xr[:, :, 2].reshape(BL, 8, 128)

    s_anchor = jax.ShapeDtypeStruct((BL, _NP), jnp.float32)
    ax, ay, az = pl.pallas_call(
        _fps_body,
        out_shape=[s_anchor, s_anchor, s_anchor],
    )(x8, y8, z8)
    anchors = jnp.stack([ax, ay, az], axis=-1)

    xT = jnp.swapaxes(xr, 1, 2)
    w = params['conv_d_w']
    w3 = jnp.swapaxes(w[:, :3], 0, 1)
    w4 = w[:, 3].reshape(1, _D)

    def fmap(di):
        def imap(g):
            b = g // L
            t = g % L
            return (b * L + jnp.clip(t + di, 0, L - 1), 0, 0)
        return imap

    feats = pl.pallas_call(
        _group_body,
        grid=(BL,),
        in_specs=[
            pl.BlockSpec((1, _NP, 3), lambda g: (g, 0, 0)),
            pl.BlockSpec((1, N, 3), fmap(-1)),
            pl.BlockSpec((1, N, 3), fmap(0)),
            pl.BlockSpec((1, N, 3), fmap(1)),
            pl.BlockSpec((1, 3, N), fmap(-1)),
            pl.BlockSpec((1, 3, N), fmap(0)),
            pl.BlockSpec((1, 3, N), fmap(1)),
            pl.BlockSpec((3, _D), lambda g: (0, 0)),
            pl.BlockSpec((1, _D), lambda g: (0, 0)),
        ],
        out_specs=pl.BlockSpec((1, _NP, _D), lambda g: (g, 0, 0)),
        out_shape=jax.ShapeDtypeStruct((BL, _NP, _D), jnp.float32),
    )(anchors, xr, xr, xr, xT, xT, xT, w3, w4)

    r1 = lambda a: a.reshape(1, -1)
    wargs = []
    for l in params['layers']:
        wargs += [r1(l['attn_norm_g']), r1(l['attn_norm_b']), l['w_qkv'],
                  l['w_spatial'], l['w_out'], r1(l['b_out']),
                  r1(l['ff_norm_g']), r1(l['ff_norm_b']), l['w1'],
                  r1(l['b1']), l['w2'], r1(l['b2'])]
    wargs += [r1(params['head_norm_g']), r1(params['head_norm_b']),
              params['head_w1'], r1(params['head_b1']),
              params['head_w2'], r1(params['head_b2'])]
    ncls = params['head_w2'].shape[1]

    T = L * _NP
    full = lambda arr: pl.BlockSpec(arr.shape, lambda b: (0,) * arr.ndim)
    logits = pl.pallas_call(
        _tx_body,
        grid=(B,),
        in_specs=[
            pl.BlockSpec((1, T, _D), lambda b: (b, 0, 0)),
            pl.BlockSpec((1, T, 3), lambda b: (b, 0, 0)),
        ] + [full(a) for a in wargs],
        out_specs=pl.BlockSpec((1, 1, ncls), lambda b: (b, 0, 0)),
        out_shape=jax.ShapeDtypeStruct((B, 1, ncls), jnp.float32),
    )(feats.reshape(B, T, _D), anchors.reshape(B, T, 3), *wargs)
    return logits.reshape(B, ncls)


# speedup vs baseline: 14.9210x; 1.0013x over previous
import functools

import jax
import jax.numpy as jnp
from jax.experimental import pallas as pl

_R2 = 0.2 * 0.2
_K = 32
_L = 4
_NP = 128
_N = 1024
_D = 512
_HEADS = 8
_DH = 64
_NEG = -1e30


def _dg(a, b):
    return jax.lax.dot_general(a, b, (((1,), (0,)), ((), ())),
                               preferred_element_type=jnp.float32)


def _dgt(a, b):
    return jax.lax.dot_general(a, b, (((1,), (1,)), ((), ())),
                               preferred_element_type=jnp.float32)


def _fps_body(x_ref, y_ref, z_ref, ax_ref, ay_ref, az_ref):
    x = x_ref[...]
    y = y_ref[...]
    z = z_ref[...]
    P = x.shape[0]
    lin = (jax.lax.broadcasted_iota(jnp.int32, (1, 8, 128), 1) * 128 +
           jax.lax.broadcasted_iota(jnp.int32, (1, 8, 128), 2))
    col = jax.lax.broadcasted_iota(jnp.int32, (1, _NP), 1)

    def step(s, carry):
        dists, far, ax, ay, az = carry
        m = lin == far
        cx = jnp.sum(jnp.where(m, x, 0.), axis=(1, 2), keepdims=True)
        cy = jnp.sum(jnp.where(m, y, 0.), axis=(1, 2), keepdims=True)
        cz = jnp.sum(jnp.where(m, z, 0.), axis=(1, 2), keepdims=True)
        ax = jnp.where(col == s, cx[:, :, 0], ax)
        ay = jnp.where(col == s, cy[:, :, 0], ay)
        az = jnp.where(col == s, cz[:, :, 0], az)
        d = (x - cx) ** 2 + (y - cy) ** 2 + (z - cz) ** 2
        dists = jnp.minimum(dists, d)
        mx = jnp.max(dists, axis=(1, 2), keepdims=True)
        cand = jnp.where(dists == mx, lin, jnp.int32(1 << 30))
        far = jnp.min(cand, axis=(1, 2), keepdims=True)
        return dists, far, ax, ay, az

    init = (jnp.full((P, 8, 128), 1e10, jnp.float32),
            jnp.zeros((P, 1, 1), jnp.int32),
            jnp.zeros((P, _NP), jnp.float32),
            jnp.zeros((P, _NP), jnp.float32),
            jnp.zeros((P, _NP), jnp.float32))
    _, _, ax, ay, az = jax.lax.fori_loop(0, _NP, step, init)
    ax_ref[...] = ax
    ay_ref[...] = ay
    az_ref[...] = az


def _group_body(a_ref, x0_ref, x1_ref, x2_ref, t0_ref, t1_ref, t2_ref,
                w3_ref, w4_ref, tri_ref, o_ref):
    a = a_ref[0]
    w3 = w3_ref[...]
    w4 = w4_ref[...]
    ax = a[:, 0:1]
    ay = a[:, 1:2]
    az = a[:, 2:3]
    tri = tri_ref[...]
    kio3 = jax.lax.broadcasted_iota(jnp.int32, (1, _K, 1), 1).astype(jnp.float32)
    lastcol = jax.lax.broadcasted_iota(jnp.int32, (_NP, _N), 1) == _N - 1
    arep = jnp.broadcast_to(a[:, None, :], (_NP, _K, 3)).reshape(_NP * _K, 3)

    acc = jnp.full((_NP, _D), _NEG, jnp.float32)
    for di, (xr, tr) in zip((-1, 0, 1),
                            ((x0_ref, t0_ref), (x1_ref, t1_ref),
                             (x2_ref, t2_ref))):
        p = xr[0]
        pT = tr[0]
        px = pT[0:1, :]
        py = pT[1:2, :]
        pz = pT[2:3, :]
        d2 = (ax - px) ** 2 + (ay - py) ** 2 + (az - pz) ** 2
        mask = d2 < _R2
        mf = mask.astype(jnp.float32)
        rank_incl = jax.lax.dot_general(
            mask.astype(jnp.bfloat16), tri, (((1,), (0,)), ((), ())),
            preferred_element_type=jnp.float32)
        rank_ex = rank_incl - mf
        count = rank_incl[:, _N - 1:_N]
        sel = mask & (rank_ex < float(_K))
        sel = sel | (lastcol & (count == 0.))
        ranksel = jnp.where(sel, rank_ex, -1.)
        o2 = (ranksel[:, None, :] == kio3)
        o2 = o2.astype(jnp.float32).reshape(_NP * _K, _N)
        g3 = _dg(o2, p)
        disp = g3 - arep
        fk = _dg(disp, w3).reshape(_NP, _K, _D)
        validk = kio3 < jnp.maximum(count[:, :, None], 1.)
        fi = jnp.max(jnp.where(validk, fk, _NEG), axis=1)
        acc = jnp.maximum(acc, fi + float(di) * w4)
    o_ref[0] = acc


def _ln(x, g, b):
    mu = jnp.mean(x, axis=-1, keepdims=True)
    var = jnp.mean((x - mu) ** 2, axis=-1, keepdims=True)
    return (x - mu) / jnp.sqrt(var + 1e-5) * g + b


def _tx_body(*refs):
    x_ref, pos_ref = refs[0], refs[1]
    hg_ref, hb_ref, hw1_ref, hb1_ref, hw2_ref, hb2_ref = refs[-7:-1]
    o_ref = refs[-1]
    x = x_ref[0]
    pos = pos_ref[0]
    scale = _DH ** -0.5
    for li in range(2):
        (lg_ref, lb_ref, wqkv_ref, wsp_ref, wout_ref, bout_ref, fg_ref,
         fb_ref, w1_ref, b1_ref, w2_ref, b2_ref) = refs[2 + 12 * li:
                                                        2 + 12 * (li + 1)]
        h = _ln(x, lg_ref[...], lb_ref[...])
        qkv = _dg(h, wqkv_ref[...])
        wsp = wsp_ref[...]
        outs = []
        for hh in range(_HEADS):
            q = qkv[:, hh * _DH:(hh + 1) * _DH]
            k = qkv[:, _D + hh * _DH:_D + (hh + 1) * _DH]
            v = qkv[:, 2 * _D + hh * _DH:2 * _D + (hh + 1) * _DH]
            qs = _dgt(q, wsp)
            dots = (_dgt(q, k) - _dgt(qs, pos)) * scale
            mx = jnp.max(dots, axis=-1, keepdims=True)
            e = jnp.exp(dots - mx)
            attn = e / jnp.sum(e, axis=-1, keepdims=True)
            outs.append(_dg(attn, v))
        o = jnp.concatenate(outs, axis=1)
        x = _dg(o, wout_ref[...]) + bout_ref[...] + x
        h2 = _ln(x, fg_ref[...], fb_ref[...])
        f = jax.nn.gelu(_dg(h2, w1_ref[...]) + b1_ref[...])
        x = _dg(f, w2_ref[...]) + b2_ref[...] + x
    pooled = jnp.max(x, axis=0, keepdims=True)
    hh_ = _ln(pooled, hg_ref[...], hb_ref[...])
    g1 = jax.nn.gelu(_dg(hh_, hw1_ref[...]) + hb1_ref[...])
    o_ref[0] = _dg(g1, hw2_ref[...]) + hb2_ref[...]


def kernel(input, params):
    xyzs = input
    B, L, N, _ = xyzs.shape
    BL = B * L
    xr = xyzs.reshape(BL, N, 3)
    x8 = xr[:, :, 0].reshape(BL, 8, 128)
    y8 = xr[:, :, 1].reshape(BL, 8, 128)
    z8 = xr[:, :, 2].reshape(BL, 8, 128)

    s_anchor = jax.ShapeDtypeStruct((BL, _NP), jnp.float32)
    ax, ay, az = pl.pallas_call(
        _fps_body,
        out_shape=[s_anchor, s_anchor, s_anchor],
    )(x8, y8, z8)
    anchors = jnp.stack([ax, ay, az], axis=-1)

    xT = jnp.swapaxes(xr, 1, 2)
    ar = jnp.arange(N, dtype=jnp.int32)
    tri = (ar[:, None] <= ar[None, :]).astype(jnp.bfloat16)
    w = params['conv_d_w']
    w3 = jnp.swapaxes(w[:, :3], 0, 1)
    w4 = w[:, 3].reshape(1, _D)

    def fmap(di):
        def imap(g):
            b = g // L
            t = g % L
            return (b * L + jnp.clip(t + di, 0, L - 1), 0, 0)
        return imap

    feats = pl.pallas_call(
        _group_body,
        grid=(BL,),
        in_specs=[
            pl.BlockSpec((1, _NP, 3), lambda g: (g, 0, 0)),
            pl.BlockSpec((1, N, 3), fmap(-1)),
            pl.BlockSpec((1, N, 3), fmap(0)),
            pl.BlockSpec((1, N, 3), fmap(1)),
            pl.BlockSpec((1, 3, N), fmap(-1)),
            pl.BlockSpec((1, 3, N), fmap(0)),
            pl.BlockSpec((1, 3, N), fmap(1)),
            pl.BlockSpec((3, _D), lambda g: (0, 0)),
            pl.BlockSpec((1, _D), lambda g: (0, 0)),
            pl.BlockSpec((_N, _N), lambda g: (0, 0)),
        ],
        out_specs=pl.BlockSpec((1, _NP, _D), lambda g: (g, 0, 0)),
        out_shape=jax.ShapeDtypeStruct((BL, _NP, _D), jnp.float32),
    )(anchors, xr, xr, xr, xT, xT, xT, w3, w4, tri)

    r1 = lambda a: a.reshape(1, -1)
    wargs = []
    for l in params['layers']:
        wargs += [r1(l['attn_norm_g']), r1(l['attn_norm_b']), l['w_qkv'],
                  l['w_spatial'], l['w_out'], r1(l['b_out']),
                  r1(l['ff_norm_g']), r1(l['ff_norm_b']), l['w1'],
                  r1(l['b1']), l['w2'], r1(l['b2'])]
    wargs += [r1(params['head_norm_g']), r1(params['head_norm_b']),
              params['head_w1'], r1(params['head_b1']),
              params['head_w2'], r1(params['head_b2'])]
    ncls = params['head_w2'].shape[1]

    T = L * _NP
    full = lambda arr: pl.BlockSpec(arr.shape, lambda b: (0,) * arr.ndim)
    logits = pl.pallas_call(
        _tx_body,
        grid=(B,),
        in_specs=[
            pl.BlockSpec((1, T, _D), lambda b: (b, 0, 0)),
            pl.BlockSpec((1, T, 3), lambda b: (b, 0, 0)),
        ] + [full(a) for a in wargs],
        out_specs=pl.BlockSpec((1, 1, ncls), lambda b: (b, 0, 0)),
        out_shape=jax.ShapeDtypeStruct((B, 1, ncls), jnp.float32),
    )(feats.reshape(B, T, _D), anchors.reshape(B, T, 3), *wargs)
    return logits.reshape(B, ncls)


# speedup vs baseline: 15.0939x; 1.0116x over previous
import functools

import jax
import jax.numpy as jnp
from jax.experimental import pallas as pl

_R2 = 0.2 * 0.2
_K = 32
_L = 4
_NP = 128
_N = 1024
_D = 512
_HEADS = 8
_DH = 64
_NEG = -1e30


def _dg(a, b):
    return jax.lax.dot_general(a, b, (((1,), (0,)), ((), ())),
                               preferred_element_type=jnp.float32)


def _dgt(a, b):
    return jax.lax.dot_general(a, b, (((1,), (1,)), ((), ())),
                               preferred_element_type=jnp.float32)


def _fps_body(x_ref, y_ref, z_ref, ax_ref, ay_ref, az_ref):
    x = x_ref[...]
    y = y_ref[...]
    z = z_ref[...]
    P = x.shape[0]
    lin = jax.lax.broadcasted_iota(jnp.int32, (1, _N), 1)
    col = jax.lax.broadcasted_iota(jnp.int32, (1, _NP), 1)

    def step(s, carry):
        dists, far, ax, ay, az = carry
        m = lin == far
        cx = jnp.sum(jnp.where(m, x, 0.), axis=1, keepdims=True)
        cy = jnp.sum(jnp.where(m, y, 0.), axis=1, keepdims=True)
        cz = jnp.sum(jnp.where(m, z, 0.), axis=1, keepdims=True)
        ax = jnp.where(col == s, cx, ax)
        ay = jnp.where(col == s, cy, ay)
        az = jnp.where(col == s, cz, az)
        d = (x - cx) ** 2 + (y - cy) ** 2 + (z - cz) ** 2
        dists = jnp.minimum(dists, d)
        mx = jnp.max(dists, axis=1, keepdims=True)
        cand = jnp.where(dists == mx, lin, jnp.int32(1 << 30))
        far = jnp.min(cand, axis=1, keepdims=True)
        return dists, far, ax, ay, az

    init = (jnp.full((P, _N), 1e10, jnp.float32),
            jnp.zeros((P, 1), jnp.int32),
            jnp.zeros((P, _NP), jnp.float32),
            jnp.zeros((P, _NP), jnp.float32),
            jnp.zeros((P, _NP), jnp.float32))
    _, _, ax, ay, az = jax.lax.fori_loop(0, _NP, step, init)
    ax_ref[...] = ax
    ay_ref[...] = ay
    az_ref[...] = az


def _group_body(a_ref, x0_ref, x1_ref, x2_ref, t0_ref, t1_ref, t2_ref,
                w3_ref, w4_ref, tri_ref, o_ref):
    a = a_ref[0]
    w3 = w3_ref[...]
    w4 = w4_ref[...]
    ax = a[:, 0:1]
    ay = a[:, 1:2]
    az = a[:, 2:3]
    tri = tri_ref[...]
    kio3 = jax.lax.broadcasted_iota(jnp.int32, (1, _K, 1), 1).astype(jnp.float32)
    lastcol = jax.lax.broadcasted_iota(jnp.int32, (_NP, _N), 1) == _N - 1
    arep = jnp.broadcast_to(a[:, None, :], (_NP, _K, 3)).reshape(_NP * _K, 3)

    acc = jnp.full((_NP, _D), _NEG, jnp.float32)
    for di, (xr, tr) in zip((-1, 0, 1),
                            ((x0_ref, t0_ref), (x1_ref, t1_ref),
                             (x2_ref, t2_ref))):
        p = xr[0]
        pT = tr[0]
        px = pT[0:1, :]
        py = pT[1:2, :]
        pz = pT[2:3, :]
        d2 = (ax - px) ** 2 + (ay - py) ** 2 + (az - pz) ** 2
        mask = d2 < _R2
        mf = mask.astype(jnp.float32)
        rank_incl = jax.lax.dot_general(
            mask.astype(jnp.bfloat16), tri, (((1,), (0,)), ((), ())),
            preferred_element_type=jnp.float32)
        rank_ex = rank_incl - mf
        count = rank_incl[:, _N - 1:_N]
        sel = mask & (rank_ex < float(_K))
        sel = sel | (lastcol & (count == 0.))
        ranksel = jnp.where(sel, rank_ex, -1.)
        o2 = (ranksel[:, None, :] == kio3)
        o2 = o2.astype(jnp.float32).reshape(_NP * _K, _N)
        g3 = _dg(o2, p)
        disp = g3 - arep
        fk = _dg(disp, w3).reshape(_NP, _K, _D)
        validk = kio3 < jnp.maximum(count[:, :, None], 1.)
        fi = jnp.max(jnp.where(validk, fk, _NEG), axis=1)
        acc = jnp.maximum(acc, fi + float(di) * w4)
    o_ref[0] = acc


def _ln(x, g, b):
    mu = jnp.mean(x, axis=-1, keepdims=True)
    var = jnp.mean((x - mu) ** 2, axis=-1, keepdims=True)
    return (x - mu) / jnp.sqrt(var + 1e-5) * g + b


def _tx_body(*refs):
    x_ref, pos_ref = refs[0], refs[1]
    hg_ref, hb_ref, hw1_ref, hb1_ref, hw2_ref, hb2_ref = refs[-7:-1]
    o_ref = refs[-1]
    x = x_ref[0]
    pos = pos_ref[0]
    scale = _DH ** -0.5
    for li in range(2):
        (lg_ref, lb_ref, wqkv_ref, wsp_ref, wout_ref, bout_ref, fg_ref,
         fb_ref, w1_ref, b1_ref, w2_ref, b2_ref) = refs[2 + 12 * li:
                                                        2 + 12 * (li + 1)]
        h = _ln(x, lg_ref[...], lb_ref[...])
        qkv = _dg(h, wqkv_ref[...])
        wsp = wsp_ref[...]
        outs = []
        for hh in range(_HEADS):
            q = qkv[:, hh * _DH:(hh + 1) * _DH]
            k = qkv[:, _D + hh * _DH:_D + (hh + 1) * _DH]
            v = qkv[:, 2 * _D + hh * _DH:2 * _D + (hh + 1) * _DH]
            qs = _dgt(q, wsp)
            dots = (_dgt(q, k) - _dgt(qs, pos)) * scale
            mx = jnp.max(dots, axis=-1, keepdims=True)
            e = jnp.exp(dots - mx)
            attn = e / jnp.sum(e, axis=-1, keepdims=True)
            outs.append(_dg(attn, v))
        o = jnp.concatenate(outs, axis=1)
        x = _dg(o, wout_ref[...]) + bout_ref[...] + x
        h2 = _ln(x, fg_ref[...], fb_ref[...])
        f = jax.nn.gelu(_dg(h2, w1_ref[...]) + b1_ref[...])
        x = _dg(f, w2_ref[...]) + b2_ref[...] + x
    pooled = jnp.max(x, axis=0, keepdims=True)
    hh_ = _ln(pooled, hg_ref[...], hb_ref[...])
    g1 = jax.nn.gelu(_dg(hh_, hw1_ref[...]) + hb1_ref[...])
    o_ref[0] = _dg(g1, hw2_ref[...]) + hb2_ref[...]


def kernel(input, params):
    xyzs = input
    B, L, N, _ = xyzs.shape
    BL = B * L
    xr = xyzs.reshape(BL, N, 3)
    x8 = xr[:, :, 0]
    y8 = xr[:, :, 1]
    z8 = xr[:, :, 2]

    s_anchor = jax.ShapeDtypeStruct((BL, _NP), jnp.float32)
    ax, ay, az = pl.pallas_call(
        _fps_body,
        out_shape=[s_anchor, s_anchor, s_anchor],
    )(x8, y8, z8)
    anchors = jnp.stack([ax, ay, az], axis=-1)

    xT = jnp.swapaxes(xr, 1, 2)
    ar = jnp.arange(N, dtype=jnp.int32)
    tri = (ar[:, None] <= ar[None, :]).astype(jnp.bfloat16)
    w = params['conv_d_w']
    w3 = jnp.swapaxes(w[:, :3], 0, 1)
    w4 = w[:, 3].reshape(1, _D)

    def fmap(di):
        def imap(g):
            b = g // L
            t = g % L
            return (b * L + jnp.clip(t + di, 0, L - 1), 0, 0)
        return imap

    feats = pl.pallas_call(
        _group_body,
        grid=(BL,),
        in_specs=[
            pl.BlockSpec((1, _NP, 3), lambda g: (g, 0, 0)),
            pl.BlockSpec((1, N, 3), fmap(-1)),
            pl.BlockSpec((1, N, 3), fmap(0)),
            pl.BlockSpec((1, N, 3), fmap(1)),
            pl.BlockSpec((1, 3, N), fmap(-1)),
            pl.BlockSpec((1, 3, N), fmap(0)),
            pl.BlockSpec((1, 3, N), fmap(1)),
            pl.BlockSpec((3, _D), lambda g: (0, 0)),
            pl.BlockSpec((1, _D), lambda g: (0, 0)),
            pl.BlockSpec((_N, _N), lambda g: (0, 0)),
        ],
        out_specs=pl.BlockSpec((1, _NP, _D), lambda g: (g, 0, 0)),
        out_shape=jax.ShapeDtypeStruct((BL, _NP, _D), jnp.float32),
    )(anchors, xr, xr, xr, xT, xT, xT, w3, w4, tri)

    r1 = lambda a: a.reshape(1, -1)
    wargs = []
    for l in params['layers']:
        wargs += [r1(l['attn_norm_g']), r1(l['attn_norm_b']), l['w_qkv'],
                  l['w_spatial'], l['w_out'], r1(l['b_out']),
                  r1(l['ff_norm_g']), r1(l['ff_norm_b']), l['w1'],
                  r1(l['b1']), l['w2'], r1(l['b2'])]
    wargs += [r1(params['head_norm_g']), r1(params['head_norm_b']),
              params['head_w1'], r1(params['head_b1']),
              params['head_w2'], r1(params['head_b2'])]
    ncls = params['head_w2'].shape[1]

    T = L * _NP
    full = lambda arr: pl.BlockSpec(arr.shape, lambda b: (0,) * arr.ndim)
    logits = pl.pallas_call(
        _tx_body,
        grid=(B,),
        in_specs=[
            pl.BlockSpec((1, T, _D), lambda b: (b, 0, 0)),
            pl.BlockSpec((1, T, 3), lambda b: (b, 0, 0)),
        ] + [full(a) for a in wargs],
        out_specs=pl.BlockSpec((1, 1, ncls), lambda b: (b, 0, 0)),
        out_shape=jax.ShapeDtypeStruct((B, 1, ncls), jnp.float32),
    )(feats.reshape(B, T, _D), anchors.reshape(B, T, 3), *wargs)
    return logits.reshape(B, ncls)
